# baseline (speedup 1.0000x reference)
import jax
import jax.numpy as jnp
from jax import lax
from jax.experimental import pallas as pl


_ROWS_PER_BLK = 1024
_FULL = 1000
_NC = 3


def _body(pre_ref, lab_ref, score_ref, pred_oh_ref, lab_oh_ref):
    x = pre_ref[:, :128]
    lane = lax.broadcasted_iota(jnp.int32, x.shape, 1)
    valid = lane < _NC
    neg_inf = jnp.float32(-jnp.inf)
    xm = jnp.where(valid, x, neg_inf)
    m = jnp.max(xm, axis=1, keepdims=True)
    e = jnp.where(valid, jnp.exp(x - m), 0.0)
    s = jnp.sum(e, axis=1, keepdims=True)
    y = e / s

    big = jnp.int32(10**6)
    idx = jnp.where(valid & (xm == m), lane, big)
    pred = jnp.min(idx, axis=1, keepdims=True)

    score_ref[...] = y
    pred_oh_ref[...] = ((lane == pred) & valid).astype(jnp.float32)
    lab = lab_ref[...]
    lab_oh_ref[...] = ((lane == lab) & valid).astype(jnp.float32)


def kernel(pre, y_label, stage_name):
    n, _ = pre.shape
    grid = n // _ROWS_PER_BLK
    lab2d = y_label.reshape(n, 1).astype(jnp.int32)
    out_shapes = (
        jax.ShapeDtypeStruct((n, 128), jnp.float32),
        jax.ShapeDtypeStruct((n, 128), jnp.float32),
        jax.ShapeDtypeStruct((n, 128), jnp.float32),
    )
    o_spec = pl.BlockSpec((_ROWS_PER_BLK, 128), lambda i: (i, 0))
    score, pred_oh, lab_oh = pl.pallas_call(
        _body,
        grid=(grid,),
        in_specs=[
            pl.BlockSpec((_ROWS_PER_BLK, _FULL), lambda i: (i, 0)),
            pl.BlockSpec((_ROWS_PER_BLK, 1), lambda i: (i, 0)),
        ],
        out_specs=(o_spec, o_spec, o_spec),
        out_shape=out_shapes,
    )(pre, lab2d)
    return (score[:, :_NC], pred_oh[:, :_NC], lab_oh[:, :_NC])


# speedup vs baseline: 1.1645x; 1.1645x over previous
import jax
import jax.numpy as jnp
from jax import lax
from jax.experimental import pallas as pl


_R = 1024
_WAYS = 4
_NC = 3


def _compute(x, lane, valid, lab, score_ref, pred_oh_ref, lab_oh_ref, k):
    neg_inf = jnp.float32(-jnp.inf)
    xm = jnp.where(valid, x, neg_inf)
    m = jnp.max(xm, axis=1, keepdims=True)
    e = jnp.where(valid, jnp.exp(x - m), 0.0)
    s = jnp.sum(e, axis=1, keepdims=True)
    y = e / s

    big = jnp.int32(10**6)
    idx = jnp.where(valid & (xm == m), lane, big)
    pred = jnp.min(idx, axis=1, keepdims=True)

    lane3 = lax.broadcasted_iota(jnp.int32, (_R, _NC), 1)
    rsl = pl.ds(k * _R, _R)
    score_ref[rsl, :] = y[:, :_NC]
    pred_oh_ref[rsl, :] = (lane3 == pred).astype(jnp.float32)
    lab_oh_ref[rsl, :] = (lane3 == lab).astype(jnp.float32)


def _body(p0, p1, p2, p3, lab_ref, score_ref, pred_oh_ref, lab_oh_ref):
    lane = lax.broadcasted_iota(jnp.int32, (_R, 128), 1)
    valid = lane < _NC
    for k, p in enumerate((p0, p1, p2, p3)):
        lab = lab_ref[pl.ds(k * _R, _R), :]
        _compute(p[...], lane, valid, lab, score_ref, pred_oh_ref,
                 lab_oh_ref, k)


def kernel(pre, y_label, stage_name):
    n, _ = pre.shape
    rows_step = _R * _WAYS
    grid = n // rows_step
    lab2d = y_label.reshape(n, 1).astype(jnp.int32)
    out_shapes = (
        jax.ShapeDtypeStruct((n, _NC), jnp.float32),
        jax.ShapeDtypeStruct((n, _NC), jnp.float32),
        jax.ShapeDtypeStruct((n, _NC), jnp.float32),
    )
    o_spec = pl.BlockSpec((rows_step, _NC), lambda i: (i, 0))

    def in_spec(k):
        return pl.BlockSpec((_R, 128), lambda i, k=k: (i * _WAYS + k, 0))

    return pl.pallas_call(
        _body,
        grid=(grid,),
        in_specs=[in_spec(0), in_spec(1), in_spec(2), in_spec(3),
                  pl.BlockSpec((rows_step, 1), lambda i: (i, 0))],
        out_specs=(o_spec, o_spec, o_spec),
        out_shape=out_shapes,
    )(pre, pre, pre, pre, lab2d)
